# baseline (speedup 1.0000x reference)
import functools

import jax
import jax.numpy as jnp
from jax import lax
from jax.experimental import pallas as pl
from jax.experimental.pallas import tpu as pltpu
from jax.experimental.pallas import tpu_sc as plsc

N = 10000
E = 320000
D_IN = 128
DH = 256
DHALF = 128
D_T = 4

NC = 2
NS = 16
CHUNK = 128
EDGES_PER_SUB = E // NS
NCHUNKS = EDGES_PER_SUB // CHUNK
TAIL = EDGES_PER_SUB - NCHUNKS * CHUNK
RSTRIPE = 624
RSTRIPE_LAST = N - (NS - 1) * RSTRIPE

ROWBLK = 1000
GRID = N // ROWBLK


def _enc_body(f_ref, w_ref, b_ref, h2_ref):
    h = jnp.maximum(f_ref[...] @ w_ref[...] + b_ref[...], 0.0)
    h2_ref[0] = h[:, :DHALF]
    h2_ref[1] = h[:, DHALF:]


def _encode(feature, W_enc, b_enc):
    return pl.pallas_call(
        _enc_body,
        grid=(GRID,),
        in_specs=[
            pl.BlockSpec((ROWBLK, D_IN), lambda i: (i, 0)),
            pl.BlockSpec((D_IN, DH), lambda i: (0, 0)),
            pl.BlockSpec((1, DH), lambda i: (0, 0)),
        ],
        out_specs=pl.BlockSpec((NC, ROWBLK, DHALF), lambda i: (0, i, 0)),
        out_shape=jax.ShapeDtypeStruct((NC, N, DHALF), jnp.float32),
    )(feature, W_enc, b_enc.reshape(1, DH))


def _sc_body(h_hbm, src_hbm, dst_hbm, z_hbm, out_hbm,
             srcv, dstv, rows, srcv_t, dstv_t, acc_sh,
             isems, idems, gsems, ssems):
    c = lax.axis_index("c")
    s = lax.axis_index("s")
    hc = h_hbm.at[c]
    ebase = s * EDGES_PER_SUB

    rbase = s * RSTRIPE

    @pl.when(s < NS - 1)
    def _():
        pltpu.sync_copy(z_hbm.at[pl.ds(rbase, RSTRIPE)],
                        acc_sh.at[pl.ds(rbase, RSTRIPE)])

    @pl.when(s == NS - 1)
    def _():
        pltpu.sync_copy(z_hbm.at[pl.ds(rbase, RSTRIPE_LAST)],
                        acc_sh.at[pl.ds(rbase, RSTRIPE_LAST)])

    plsc.subcore_barrier()

    def idx_load(i, p):
        pltpu.async_copy(src_hbm.at[pl.ds(ebase + i * CHUNK, CHUNK)],
                         srcv[p], isems[p])
        pltpu.async_copy(dst_hbm.at[pl.ds(ebase + i * CHUNK, CHUNK)],
                         dstv[p], idems[p])

    def idx_drain(p):
        pltpu.make_async_copy(src_hbm.at[pl.ds(0, CHUNK)], srcv[p],
                              isems[p]).wait()
        pltpu.make_async_copy(dst_hbm.at[pl.ds(0, CHUNK)], dstv[p],
                              idems[p]).wait()

    def gather_drain(p):
        pltpu.make_async_copy(hc.at[srcv[p]], rows[p], gsems[p]).wait()

    def scatter_drain(p):
        pltpu.make_async_copy(rows[p], acc_sh.at[dstv[p]], ssems[p]).wait()

    idx_load(0, 0)
    idx_load(1, 1)
    idx_load(2, 2)
    idx_drain(0)
    pltpu.async_copy(hc.at[srcv[0]], rows[0], gsems[0])

    def slot_step(i, p):
        pn = (p + 1) % 3
        pq = (p + 2) % 3

        @pl.when(i < NCHUNKS - 1)
        def _():
            idx_drain(pn)
            pltpu.async_copy(hc.at[srcv[pn]], rows[pn], gsems[pn])

        gather_drain(p)
        pltpu.async_copy(rows[p], acc_sh.at[dstv[p]], ssems[p], add=True)

        @pl.when(i >= 1)
        def _():
            scatter_drain(pq)

        @pl.when(jnp.logical_and(i >= 1, i <= NCHUNKS - 3))
        def _():
            idx_load(i + 2, pq)

    def body(i, carry):
        r = lax.rem(i, 3)

        @pl.when(r == 0)
        def _():
            slot_step(i, 0)

        @pl.when(r == 1)
        def _():
            slot_step(i, 1)

        @pl.when(r == 2)
        def _():
            slot_step(i, 2)

        return carry

    lax.fori_loop(0, NCHUNKS, body, 0)
    scatter_drain((NCHUNKS - 1) % 3)

    tbase = ebase + NCHUNKS * CHUNK
    rows_t = rows[0].at[pl.ds(0, TAIL)]
    pltpu.sync_copy(src_hbm.at[pl.ds(tbase, TAIL)], srcv_t)
    pltpu.sync_copy(dst_hbm.at[pl.ds(tbase, TAIL)], dstv_t)
    pltpu.async_copy(hc.at[srcv_t], rows_t, gsems[0]).wait()
    pltpu.async_copy(rows_t, acc_sh.at[dstv_t], ssems[0], add=True).wait()
    plsc.subcore_barrier()

    @pl.when(s < NS - 1)
    def _():
        pltpu.sync_copy(acc_sh.at[pl.ds(rbase, RSTRIPE)],
                        out_hbm.at[c].at[pl.ds(rbase, RSTRIPE)])

    @pl.when(s == NS - 1)
    def _():
        pltpu.sync_copy(acc_sh.at[pl.ds(rbase, RSTRIPE_LAST)],
                        out_hbm.at[c].at[pl.ds(rbase, RSTRIPE_LAST)])


def _segment_sum(h2, src, dst):
    zeros = jnp.zeros((N, DHALF), jnp.float32)
    mesh = plsc.VectorSubcoreMesh(core_axis_name="c", subcore_axis_name="s")
    return pl.kernel(
        _sc_body,
        out_type=jax.ShapeDtypeStruct((NC, N, DHALF), jnp.float32),
        mesh=mesh,
        scratch_types=[
            [pltpu.VMEM((CHUNK,), jnp.int32) for _ in range(3)],
            [pltpu.VMEM((CHUNK,), jnp.int32) for _ in range(3)],
            [pltpu.VMEM((CHUNK, DHALF), jnp.float32) for _ in range(3)],
            pltpu.VMEM((TAIL,), jnp.int32),
            pltpu.VMEM((TAIL,), jnp.int32),
            pltpu.VMEM_SHARED((N, DHALF), jnp.float32),
            [pltpu.SemaphoreType.DMA for _ in range(3)],
            [pltpu.SemaphoreType.DMA for _ in range(3)],
            [pltpu.SemaphoreType.DMA for _ in range(3)],
            [pltpu.SemaphoreType.DMA for _ in range(3)],
        ],
    )(h2, src, dst, zeros)


def _mlp_body(h2_ref, agg_ref, wm1_ref, bm1_ref, wm2_ref, bm2_ref,
              wa_ref, ba_ref, wb_ref, bb_ref, wc_ref, bc_ref,
              hout_ref, a_ref):
    x = jnp.concatenate([h2_ref[0] + agg_ref[0], h2_ref[1] + agg_ref[1]],
                        axis=1)
    m = jnp.maximum(x @ wm1_ref[...] + bm1_ref[...], 0.0)
    h = m @ wm2_ref[...] + bm2_ref[...]
    a = jnp.tanh(h @ wa_ref[...] + ba_ref[...])
    g = jax.nn.sigmoid(h @ wb_ref[...] + bb_ref[...])
    hout_ref[...] = h
    a_ref[...] = (a * g) @ wc_ref[...] + bc_ref[...]


def _mlp_attn(h2, agg2, W_m1, b_m1, W_m2, b_m2, W_a, b_a, W_b, b_b, W_c, b_c):
    full = lambda r, c: pl.BlockSpec((r, c), lambda i: (0, 0))
    return pl.pallas_call(
        _mlp_body,
        grid=(GRID,),
        in_specs=[
            pl.BlockSpec((NC, ROWBLK, DHALF), lambda i: (0, i, 0)),
            pl.BlockSpec((NC, ROWBLK, DHALF), lambda i: (0, i, 0)),
            full(DH, DH), full(1, DH), full(DH, DH), full(1, DH),
            full(DH, DH), full(1, DH), full(DH, DH), full(1, DH),
            full(DH, 1), full(1, 1),
        ],
        out_specs=[
            pl.BlockSpec((ROWBLK, DH), lambda i: (i, 0)),
            pl.BlockSpec((ROWBLK, 1), lambda i: (i, 0)),
        ],
        out_shape=[
            jax.ShapeDtypeStruct((N, DH), jnp.float32),
            jax.ShapeDtypeStruct((N, 1), jnp.float32),
        ],
    )(h2, agg2, W_m1, b_m1.reshape(1, DH), W_m2, b_m2.reshape(1, DH),
      W_a, b_a.reshape(1, DH), W_b, b_b.reshape(1, DH),
      W_c, b_c.reshape(1, 1))


def _pool_body(h_ref, a_ref, gamma_ref, beta_ref, wo_ref, bo_ref, out_ref):
    scores = a_ref[...][:, 0]
    amax = jnp.max(scores)
    e = jnp.exp(scores - amax)
    w = e / jnp.sum(e)
    pooled = w[None, :] @ h_ref[...]
    mu = jnp.mean(pooled)
    var = jnp.mean((pooled - mu) ** 2)
    pn = (pooled - mu) * jax.lax.rsqrt(var + 1e-5)
    pn = pn * gamma_ref[...] + beta_ref[...]
    out_ref[...] = pn @ wo_ref[...] + bo_ref[...]


def _pool_head(h, A, gamma, beta, W_out, b_out):
    return pl.pallas_call(
        _pool_body,
        out_shape=jax.ShapeDtypeStruct((1, D_T), jnp.float32),
    )(h, A, gamma.reshape(1, DH), beta.reshape(1, DH),
      W_out, b_out.reshape(1, D_T))


def kernel(feature, edge_index, batch, W_enc, b_enc, W_m1, b_m1, W_m2, b_m2,
           W_a, b_a, W_b, b_b, W_c, b_c, gamma, beta, W_out, b_out):
    src = edge_index[0]
    dst = edge_index[1]
    h2 = _encode(feature, W_enc, b_enc)
    agg2 = _segment_sum(h2, src, dst)
    h, A = _mlp_attn(h2, agg2, W_m1, b_m1, W_m2, b_m2,
                     W_a, b_a, W_b, b_b, W_c, b_c)
    return _pool_head(h, A, gamma, beta, W_out, b_out)


# speedup vs baseline: 1.0686x; 1.0686x over previous
import functools

import jax
import jax.numpy as jnp
from jax import lax
from jax.experimental import pallas as pl
from jax.experimental.pallas import tpu as pltpu
from jax.experimental.pallas import tpu_sc as plsc

N = 10000
E = 320000
D_IN = 128
DH = 256
DHALF = 128
D_T = 4

NC = 2
NS = 16
CHUNK = 96
EDGES_PER_SUB = E // NS
NCHUNKS = EDGES_PER_SUB // CHUNK
TAIL = EDGES_PER_SUB - NCHUNKS * CHUNK
RSTRIPE = 624
RSTRIPE_LAST = N - (NS - 1) * RSTRIPE

ROWBLK = 1000
GRID = N // ROWBLK


def _enc_body(f_ref, w_ref, b_ref, h2_ref):
    h = jnp.maximum(f_ref[...] @ w_ref[...] + b_ref[...], 0.0)
    h2_ref[0] = h[:, :DHALF]
    h2_ref[1] = h[:, DHALF:]


def _encode(feature, W_enc, b_enc):
    return pl.pallas_call(
        _enc_body,
        grid=(GRID,),
        in_specs=[
            pl.BlockSpec((ROWBLK, D_IN), lambda i: (i, 0)),
            pl.BlockSpec((D_IN, DH), lambda i: (0, 0)),
            pl.BlockSpec((1, DH), lambda i: (0, 0)),
        ],
        out_specs=pl.BlockSpec((NC, ROWBLK, DHALF), lambda i: (0, i, 0)),
        out_shape=jax.ShapeDtypeStruct((NC, N, DHALF), jnp.float32),
    )(feature, W_enc, b_enc.reshape(1, DH))


def _sc_body(h_hbm, src_hbm, dst_hbm, z_hbm, out_hbm,
             srcv, dstv, rows, srcv_t, dstv_t, acc_sh,
             isems, idems, gsems, ssems):
    c = lax.axis_index("c")
    s = lax.axis_index("s")
    hc = h_hbm.at[c]
    ebase = s * EDGES_PER_SUB

    rbase = s * RSTRIPE

    @pl.when(s < NS - 1)
    def _():
        pltpu.sync_copy(z_hbm.at[pl.ds(rbase, RSTRIPE)],
                        acc_sh.at[pl.ds(rbase, RSTRIPE)])

    @pl.when(s == NS - 1)
    def _():
        pltpu.sync_copy(z_hbm.at[pl.ds(rbase, RSTRIPE_LAST)],
                        acc_sh.at[pl.ds(rbase, RSTRIPE_LAST)])

    plsc.subcore_barrier()

    def idx_load(i, p):
        pltpu.async_copy(src_hbm.at[pl.ds(ebase + i * CHUNK, CHUNK)],
                         srcv[p], isems[p])
        pltpu.async_copy(dst_hbm.at[pl.ds(ebase + i * CHUNK, CHUNK)],
                         dstv[p], idems[p])

    def idx_drain(p):
        pltpu.make_async_copy(src_hbm.at[pl.ds(0, CHUNK)], srcv[p],
                              isems[p]).wait()
        pltpu.make_async_copy(dst_hbm.at[pl.ds(0, CHUNK)], dstv[p],
                              idems[p]).wait()

    def gather_drain(p):
        pltpu.make_async_copy(hc.at[srcv[p]], rows[p], gsems[p]).wait()

    def scatter_drain(p):
        pltpu.make_async_copy(rows[p], acc_sh.at[dstv[p]], ssems[p]).wait()

    idx_load(0, 0)
    idx_load(1, 1)
    idx_load(2, 2)
    idx_load(3, 3)
    idx_drain(0)
    pltpu.async_copy(hc.at[srcv[0]], rows[0], gsems[0])
    idx_drain(1)
    pltpu.async_copy(hc.at[srcv[1]], rows[1], gsems[1])

    def slot_step(i, p):
        pn2 = (p + 2) % 4
        pq = (p + 3) % 4

        @pl.when(i < NCHUNKS - 2)
        def _():
            idx_drain(pn2)
            pltpu.async_copy(hc.at[srcv[pn2]], rows[pn2], gsems[pn2])

        gather_drain(p)
        pltpu.async_copy(rows[p], acc_sh.at[dstv[p]], ssems[p], add=True)

        @pl.when(i >= 1)
        def _():
            scatter_drain(pq)

        @pl.when(jnp.logical_and(i >= 1, i <= NCHUNKS - 4))
        def _():
            idx_load(i + 3, pq)

    def body(i, carry):
        r = lax.rem(i, 4)

        @pl.when(r == 0)
        def _():
            slot_step(i, 0)

        @pl.when(r == 1)
        def _():
            slot_step(i, 1)

        @pl.when(r == 2)
        def _():
            slot_step(i, 2)

        @pl.when(r == 3)
        def _():
            slot_step(i, 3)

        return carry

    lax.fori_loop(0, NCHUNKS, body, 0)
    scatter_drain((NCHUNKS - 1) % 4)

    tbase = ebase + NCHUNKS * CHUNK
    rows_t = rows[0].at[pl.ds(0, TAIL)]
    pltpu.sync_copy(src_hbm.at[pl.ds(tbase, TAIL)], srcv_t)
    pltpu.sync_copy(dst_hbm.at[pl.ds(tbase, TAIL)], dstv_t)
    pltpu.async_copy(hc.at[srcv_t], rows_t, gsems[0]).wait()
    pltpu.async_copy(rows_t, acc_sh.at[dstv_t], ssems[0], add=True).wait()
    plsc.subcore_barrier()

    @pl.when(s < NS - 1)
    def _():
        pltpu.sync_copy(acc_sh.at[pl.ds(rbase, RSTRIPE)],
                        out_hbm.at[c].at[pl.ds(rbase, RSTRIPE)])

    @pl.when(s == NS - 1)
    def _():
        pltpu.sync_copy(acc_sh.at[pl.ds(rbase, RSTRIPE_LAST)],
                        out_hbm.at[c].at[pl.ds(rbase, RSTRIPE_LAST)])


def _segment_sum(h2, src, dst):
    zeros = jnp.zeros((N, DHALF), jnp.float32)
    mesh = plsc.VectorSubcoreMesh(core_axis_name="c", subcore_axis_name="s")
    return pl.kernel(
        _sc_body,
        out_type=jax.ShapeDtypeStruct((NC, N, DHALF), jnp.float32),
        mesh=mesh,
        scratch_types=[
            [pltpu.VMEM((CHUNK,), jnp.int32) for _ in range(4)],
            [pltpu.VMEM((CHUNK,), jnp.int32) for _ in range(4)],
            [pltpu.VMEM((CHUNK, DHALF), jnp.float32) for _ in range(4)],
            pltpu.VMEM((TAIL,), jnp.int32),
            pltpu.VMEM((TAIL,), jnp.int32),
            pltpu.VMEM_SHARED((N, DHALF), jnp.float32),
            [pltpu.SemaphoreType.DMA for _ in range(4)],
            [pltpu.SemaphoreType.DMA for _ in range(4)],
            [pltpu.SemaphoreType.DMA for _ in range(4)],
            [pltpu.SemaphoreType.DMA for _ in range(4)],
        ],
    )(h2, src, dst, zeros)


def _mlp_body(h2_ref, agg_ref, wm1_ref, bm1_ref, wm2_ref, bm2_ref,
              wa_ref, ba_ref, wb_ref, bb_ref, wc_ref, bc_ref,
              hout_ref, a_ref):
    x = jnp.concatenate([h2_ref[0] + agg_ref[0], h2_ref[1] + agg_ref[1]],
                        axis=1)
    m = jnp.maximum(x @ wm1_ref[...] + bm1_ref[...], 0.0)
    h = m @ wm2_ref[...] + bm2_ref[...]
    a = jnp.tanh(h @ wa_ref[...] + ba_ref[...])
    g = jax.nn.sigmoid(h @ wb_ref[...] + bb_ref[...])
    hout_ref[...] = h
    a_ref[...] = (a * g) @ wc_ref[...] + bc_ref[...]


def _mlp_attn(h2, agg2, W_m1, b_m1, W_m2, b_m2, W_a, b_a, W_b, b_b, W_c, b_c):
    full = lambda r, c: pl.BlockSpec((r, c), lambda i: (0, 0))
    return pl.pallas_call(
        _mlp_body,
        grid=(GRID,),
        in_specs=[
            pl.BlockSpec((NC, ROWBLK, DHALF), lambda i: (0, i, 0)),
            pl.BlockSpec((NC, ROWBLK, DHALF), lambda i: (0, i, 0)),
            full(DH, DH), full(1, DH), full(DH, DH), full(1, DH),
            full(DH, DH), full(1, DH), full(DH, DH), full(1, DH),
            full(DH, 1), full(1, 1),
        ],
        out_specs=[
            pl.BlockSpec((ROWBLK, DH), lambda i: (i, 0)),
            pl.BlockSpec((ROWBLK, 1), lambda i: (i, 0)),
        ],
        out_shape=[
            jax.ShapeDtypeStruct((N, DH), jnp.float32),
            jax.ShapeDtypeStruct((N, 1), jnp.float32),
        ],
    )(h2, agg2, W_m1, b_m1.reshape(1, DH), W_m2, b_m2.reshape(1, DH),
      W_a, b_a.reshape(1, DH), W_b, b_b.reshape(1, DH),
      W_c, b_c.reshape(1, 1))


def _pool_body(h_ref, a_ref, gamma_ref, beta_ref, wo_ref, bo_ref, out_ref):
    scores = a_ref[...][:, 0]
    amax = jnp.max(scores)
    e = jnp.exp(scores - amax)
    w = e / jnp.sum(e)
    pooled = w[None, :] @ h_ref[...]
    mu = jnp.mean(pooled)
    var = jnp.mean((pooled - mu) ** 2)
    pn = (pooled - mu) * jax.lax.rsqrt(var + 1e-5)
    pn = pn * gamma_ref[...] + beta_ref[...]
    out_ref[...] = pn @ wo_ref[...] + bo_ref[...]


def _pool_head(h, A, gamma, beta, W_out, b_out):
    return pl.pallas_call(
        _pool_body,
        out_shape=jax.ShapeDtypeStruct((1, D_T), jnp.float32),
    )(h, A, gamma.reshape(1, DH), beta.reshape(1, DH),
      W_out, b_out.reshape(1, D_T))


def kernel(feature, edge_index, batch, W_enc, b_enc, W_m1, b_m1, W_m2, b_m2,
           W_a, b_a, W_b, b_b, W_c, b_c, gamma, beta, W_out, b_out):
    src = edge_index[0]
    dst = edge_index[1]
    h2 = _encode(feature, W_enc, b_enc)
    agg2 = _segment_sum(h2, src, dst)
    h, A = _mlp_attn(h2, agg2, W_m1, b_m1, W_m2, b_m2,
                     W_a, b_a, W_b, b_b, W_c, b_c)
    return _pool_head(h, A, gamma, beta, W_out, b_out)


# speedup vs baseline: 1.1038x; 1.0329x over previous
import functools

import jax
import jax.numpy as jnp
from jax import lax
from jax.experimental import pallas as pl
from jax.experimental.pallas import tpu as pltpu
from jax.experimental.pallas import tpu_sc as plsc

N = 10000
E = 320000
D_IN = 128
DH = 256
DHALF = 128
D_T = 4

NC = 2
NS = 16
CHUNK = 96
EDGES_PER_SUB = E // NS
NCHUNKS = EDGES_PER_SUB // CHUNK
TAIL = EDGES_PER_SUB - NCHUNKS * CHUNK
RSTRIPE = 624
RSTRIPE_LAST = N - (NS - 1) * RSTRIPE

ROWBLK = 1000
GRID = N // ROWBLK


def _enc_body(f_ref, w_ref, b_ref, h2_ref):
    h = jnp.maximum(f_ref[...] @ w_ref[...] + b_ref[...], 0.0)
    h2_ref[0] = h[:, :DHALF]
    h2_ref[1] = h[:, DHALF:]


def _encode(feature, W_enc, b_enc):
    return pl.pallas_call(
        _enc_body,
        grid=(GRID,),
        in_specs=[
            pl.BlockSpec((ROWBLK, D_IN), lambda i: (i, 0)),
            pl.BlockSpec((D_IN, DH), lambda i: (0, 0)),
            pl.BlockSpec((1, DH), lambda i: (0, 0)),
        ],
        out_specs=pl.BlockSpec((NC, ROWBLK, DHALF), lambda i: (0, i, 0)),
        out_shape=jax.ShapeDtypeStruct((NC, N, DHALF), jnp.float32),
    )(feature, W_enc, b_enc.reshape(1, DH))


def _sc_body(h_hbm, src_hbm, dst_hbm, z_hbm, out_hbm,
             srcv, dstv, rows, srcv_t, dstv_t, acc_sh,
             isems, idems, gsems, ssems):
    c = lax.axis_index("c")
    s = lax.axis_index("s")
    hc = h_hbm.at[c]
    ebase = s * EDGES_PER_SUB

    rbase = s * RSTRIPE

    @pl.when(s < NS - 1)
    def _():
        pltpu.sync_copy(z_hbm.at[pl.ds(rbase, RSTRIPE)],
                        acc_sh.at[pl.ds(rbase, RSTRIPE)])

    @pl.when(s == NS - 1)
    def _():
        pltpu.sync_copy(z_hbm.at[pl.ds(rbase, RSTRIPE_LAST)],
                        acc_sh.at[pl.ds(rbase, RSTRIPE_LAST)])

    plsc.subcore_barrier()

    def idx_load(i, p):
        pltpu.async_copy(src_hbm.at[pl.ds(ebase + i * CHUNK, CHUNK)],
                         srcv[p], isems[p])
        pltpu.async_copy(dst_hbm.at[pl.ds(ebase + i * CHUNK, CHUNK)],
                         dstv[p], idems[p])

    def idx_drain(p):
        pltpu.make_async_copy(src_hbm.at[pl.ds(0, CHUNK)], srcv[p],
                              isems[p]).wait()
        pltpu.make_async_copy(dst_hbm.at[pl.ds(0, CHUNK)], dstv[p],
                              idems[p]).wait()

    def gather_drain(p):
        pltpu.make_async_copy(hc.at[srcv[p]], rows[p], gsems[p]).wait()

    def scatter_drain(p):
        pltpu.make_async_copy(rows[p], acc_sh.at[dstv[p]], ssems[p]).wait()

    idx_load(0, 0)
    idx_load(1, 1)
    idx_load(2, 2)
    idx_load(3, 3)
    idx_drain(0)
    pltpu.async_copy(hc.at[srcv[0]], rows[0], gsems[0])
    idx_drain(1)
    pltpu.async_copy(hc.at[srcv[1]], rows[1], gsems[1])

    def slot_step(i, p):
        pn2 = (p + 2) % 4
        pq = (p + 3) % 4

        @pl.when(i < NCHUNKS - 2)
        def _():
            idx_drain(pn2)
            pltpu.async_copy(hc.at[srcv[pn2]], rows[pn2], gsems[pn2])

        gather_drain(p)
        pltpu.async_copy(rows[p], acc_sh.at[dstv[p]], ssems[p], add=True)

        @pl.when(i >= 1)
        def _():
            scatter_drain(pq)

        @pl.when(jnp.logical_and(i >= 1, i <= NCHUNKS - 4))
        def _():
            idx_load(i + 3, pq)

    def body(i, carry):
        r = lax.rem(i, 4)

        @pl.when(r == 0)
        def _():
            slot_step(i, 0)

        @pl.when(r == 1)
        def _():
            slot_step(i, 1)

        @pl.when(r == 2)
        def _():
            slot_step(i, 2)

        @pl.when(r == 3)
        def _():
            slot_step(i, 3)

        return carry

    lax.fori_loop(0, NCHUNKS, body, 0)
    scatter_drain((NCHUNKS - 1) % 4)

    tbase = ebase + NCHUNKS * CHUNK
    rows_t = rows[0].at[pl.ds(0, TAIL)]
    pltpu.sync_copy(src_hbm.at[pl.ds(tbase, TAIL)], srcv_t)
    pltpu.sync_copy(dst_hbm.at[pl.ds(tbase, TAIL)], dstv_t)
    pltpu.async_copy(hc.at[srcv_t], rows_t, gsems[0]).wait()
    pltpu.async_copy(rows_t, acc_sh.at[dstv_t], ssems[0], add=True).wait()
    plsc.subcore_barrier()

    @pl.when(s < NS - 1)
    def _():
        pltpu.sync_copy(acc_sh.at[pl.ds(rbase, RSTRIPE)],
                        out_hbm.at[c].at[pl.ds(rbase, RSTRIPE)])

    @pl.when(s == NS - 1)
    def _():
        pltpu.sync_copy(acc_sh.at[pl.ds(rbase, RSTRIPE_LAST)],
                        out_hbm.at[c].at[pl.ds(rbase, RSTRIPE_LAST)])


def _segment_sum(h2, src, dst):
    zeros = jnp.zeros((N, DHALF), jnp.float32)
    mesh = plsc.VectorSubcoreMesh(core_axis_name="c", subcore_axis_name="s")
    return pl.kernel(
        _sc_body,
        out_type=jax.ShapeDtypeStruct((NC, N, DHALF), jnp.float32),
        mesh=mesh,
        scratch_types=[
            [pltpu.VMEM((CHUNK,), jnp.int32) for _ in range(4)],
            [pltpu.VMEM((CHUNK,), jnp.int32) for _ in range(4)],
            [pltpu.VMEM((CHUNK, DHALF), jnp.float32) for _ in range(4)],
            pltpu.VMEM((TAIL,), jnp.int32),
            pltpu.VMEM((TAIL,), jnp.int32),
            pltpu.VMEM_SHARED((N, DHALF), jnp.float32),
            [pltpu.SemaphoreType.DMA for _ in range(4)],
            [pltpu.SemaphoreType.DMA for _ in range(4)],
            [pltpu.SemaphoreType.DMA for _ in range(4)],
            [pltpu.SemaphoreType.DMA for _ in range(4)],
        ],
    )(h2, src, dst, zeros)


def _mlp_pool_body(h2_ref, agg_ref, wm1_ref, bm1_ref, wm2_ref, bm2_ref,
                   wa_ref, ba_ref, wb_ref, bb_ref, wc_ref, bc_ref,
                   gamma_ref, beta_ref, wo_ref, bo_ref, out_ref,
                   m_sc, s_sc, S_sc):
    i = pl.program_id(0)
    x = jnp.concatenate([h2_ref[0] + agg_ref[0], h2_ref[1] + agg_ref[1]],
                        axis=1)
    mm = jnp.maximum(x @ wm1_ref[...] + bm1_ref[...], 0.0)
    h = mm @ wm2_ref[...] + bm2_ref[...]
    a = jnp.tanh(h @ wa_ref[...] + ba_ref[...])
    g = jax.nn.sigmoid(h @ wb_ref[...] + bb_ref[...])
    scores = ((a * g) @ wc_ref[...] + bc_ref[...])[:, 0]

    @pl.when(i == 0)
    def _():
        m_sc[0, 0] = -jnp.inf
        s_sc[0, 0] = 0.0
        S_sc[...] = jnp.zeros_like(S_sc)

    m_old = m_sc[0, 0]
    m_new = jnp.maximum(m_old, jnp.max(scores))
    corr = jnp.exp(m_old - m_new)
    e = jnp.exp(scores - m_new)
    s_sc[0, 0] = s_sc[0, 0] * corr + jnp.sum(e)
    S_sc[...] = S_sc[...] * corr + e[None, :] @ h
    m_sc[0, 0] = m_new

    @pl.when(i == GRID - 1)
    def _():
        pooled = S_sc[...] / s_sc[0, 0]
        mu = jnp.mean(pooled)
        var = jnp.mean((pooled - mu) ** 2)
        pn = (pooled - mu) * jax.lax.rsqrt(var + 1e-5)
        pn = pn * gamma_ref[...] + beta_ref[...]
        out_ref[...] = pn @ wo_ref[...] + bo_ref[...]


def _mlp_pool(h2, agg2, W_m1, b_m1, W_m2, b_m2, W_a, b_a, W_b, b_b, W_c, b_c,
              gamma, beta, W_out, b_out):
    full = lambda r, c: pl.BlockSpec((r, c), lambda i: (0, 0))
    return pl.pallas_call(
        _mlp_pool_body,
        grid=(GRID,),
        in_specs=[
            pl.BlockSpec((NC, ROWBLK, DHALF), lambda i: (0, i, 0)),
            pl.BlockSpec((NC, ROWBLK, DHALF), lambda i: (0, i, 0)),
            full(DH, DH), full(1, DH), full(DH, DH), full(1, DH),
            full(DH, DH), full(1, DH), full(DH, DH), full(1, DH),
            full(DH, 1), full(1, 1),
            full(1, DH), full(1, DH), full(DH, D_T), full(1, D_T),
        ],
        out_specs=pl.BlockSpec((1, D_T), lambda i: (0, 0)),
        out_shape=jax.ShapeDtypeStruct((1, D_T), jnp.float32),
        scratch_shapes=[
            pltpu.SMEM((1, 1), jnp.float32),
            pltpu.SMEM((1, 1), jnp.float32),
            pltpu.VMEM((1, DH), jnp.float32),
        ],
    )(h2, agg2, W_m1, b_m1.reshape(1, DH), W_m2, b_m2.reshape(1, DH),
      W_a, b_a.reshape(1, DH), W_b, b_b.reshape(1, DH),
      W_c, b_c.reshape(1, 1),
      gamma.reshape(1, DH), beta.reshape(1, DH), W_out,
      b_out.reshape(1, D_T))


def kernel(feature, edge_index, batch, W_enc, b_enc, W_m1, b_m1, W_m2, b_m2,
           W_a, b_a, W_b, b_b, W_c, b_c, gamma, beta, W_out, b_out):
    src = edge_index[0]
    dst = edge_index[1]
    h2 = _encode(feature, W_enc, b_enc)
    agg2 = _segment_sum(h2, src, dst)
    return _mlp_pool(h2, agg2, W_m1, b_m1, W_m2, b_m2,
                     W_a, b_a, W_b, b_b, W_c, b_c,
                     gamma, beta, W_out, b_out)
